# Initial kernel scaffold; baseline (speedup 1.0000x reference)
import jax
import jax.numpy as jnp
from jax.experimental import pallas as pl


def kernel(range_km, sensor_params, contact_indices):
    raise NotImplementedError("write your pallas kernel here")


# speedup vs baseline: 470.1169x; 470.1169x over previous
import functools

import jax
import jax.numpy as jnp
from jax import lax
from jax.experimental import pallas as pl
from jax.experimental.pallas import tpu as pltpu
from jax.experimental.pallas import tpu_sc as plsc

N_LANES = 16
NUM_WORKERS = 32


def _gather_add_body(n_passes, per_worker, chunk, num_chunks,
                     range_hbm, params_hbm, idx_hbm, out_hbm,
                     table_v, idx_v, rng_v):
    wid = lax.axis_index("s") * 2 + lax.axis_index("c")
    base = wid * per_worker

    pltpu.sync_copy(params_hbm, table_v)

    for j in range(num_chunks):
        off = base + j * chunk
        pltpu.sync_copy(idx_hbm.at[pl.ds(off, chunk)], idx_v)
        pltpu.sync_copy(range_hbm.at[pl.ds(off, chunk)], rng_v)

        @plsc.parallel_loop(0, chunk, step=N_LANES, unroll=8)
        def _(i):
            s = pl.ds(i, N_LANES)
            vals = plsc.load_gather(table_v, [idx_v[s]])
            rng_v[s] = rng_v[s] + vals

        pltpu.sync_copy(rng_v, out_hbm.at[pl.ds(off, chunk)])


@functools.partial(jax.jit, static_argnames=())
def _radar_bias_add(range_km, sensor_params, contact_indices):
    n_meas = range_km.shape[0]
    n_passes = sensor_params.shape[0]
    assert n_meas % NUM_WORKERS == 0
    per_worker = n_meas // NUM_WORKERS
    chunk = 12800
    assert per_worker % chunk == 0
    num_chunks = per_worker // chunk

    mesh = plsc.VectorSubcoreMesh(core_axis_name="c", subcore_axis_name="s")
    body = functools.partial(_gather_add_body, n_passes, per_worker,
                             chunk, num_chunks)
    f = pl.kernel(
        body,
        out_type=jax.ShapeDtypeStruct((n_meas,), jnp.float32),
        mesh=mesh,
        compiler_params=pltpu.CompilerParams(needs_layout_passes=False),
        scratch_types=[
            pltpu.VMEM((n_passes,), jnp.float32),
            pltpu.VMEM((chunk,), jnp.int32),
            pltpu.VMEM((chunk,), jnp.float32),
        ],
    )
    return f(range_km, sensor_params, contact_indices)


def kernel(range_km, sensor_params, contact_indices):
    idx = contact_indices.astype(jnp.int32)
    return _radar_bias_add(range_km, sensor_params, idx)


# speedup vs baseline: 581.5905x; 1.2371x over previous
import functools

import jax
import jax.numpy as jnp
from jax import lax
from jax.experimental import pallas as pl
from jax.experimental.pallas import tpu as pltpu
from jax.experimental.pallas import tpu_sc as plsc

N_LANES = 16
NUM_WORKERS = 32


def _gather_add_body(per_worker, chunk, num_chunks,
                     range_hbm, params_hbm, idx_hbm, out_hbm,
                     table_v, idx_v0, idx_v1, rng_v0, rng_v1, res_v0, res_v1,
                     isem0, isem1, osem0, osem1):
    wid = lax.axis_index("s") * 2 + lax.axis_index("c")
    base = wid * per_worker
    idx_v = (idx_v0, idx_v1)
    rng_v = (rng_v0, rng_v1)
    res_v = (res_v0, res_v1)
    isems = (isem0, isem1)
    osems = (osem0, osem1)

    pltpu.sync_copy(params_hbm, table_v)

    in_copies = {}
    out_copies = {}

    def issue_in(j):
        b = j & 1
        off = base + j * chunk
        in_copies[j] = (
            pltpu.async_copy(idx_hbm.at[pl.ds(off, chunk)],
                             idx_v[b], isems[b]),
            pltpu.async_copy(range_hbm.at[pl.ds(off, chunk)],
                             rng_v[b], isems[b]),
        )

    issue_in(0)
    for j in range(num_chunks):
        b = j & 1
        if j + 1 < num_chunks:
            issue_in(j + 1)
        ci, cr = in_copies.pop(j)
        ci.wait()
        cr.wait()
        if j >= 2:
            out_copies.pop(j - 2).wait()

        idx_b = idx_v[b]
        rng_b = rng_v[b]
        res_b = res_v[b]

        @plsc.parallel_loop(0, chunk, step=N_LANES, unroll=8)
        def _(i):
            s = pl.ds(i, N_LANES)
            vals = plsc.load_gather(table_v, [idx_b[s]])
            res_b[s] = rng_b[s] + vals

        out_copies[j] = pltpu.async_copy(
            res_v[b], out_hbm.at[pl.ds(base + j * chunk, chunk)], osems[b])

    for j in sorted(out_copies):
        out_copies[j].wait()


@jax.jit
def _radar_bias_add(range_km, sensor_params, contact_indices):
    n_meas = range_km.shape[0]
    n_passes = sensor_params.shape[0]
    assert n_meas % NUM_WORKERS == 0
    per_worker = n_meas // NUM_WORKERS
    chunk = 5120
    assert per_worker % chunk == 0
    num_chunks = per_worker // chunk

    mesh = plsc.VectorSubcoreMesh(core_axis_name="c", subcore_axis_name="s")
    body = functools.partial(_gather_add_body, per_worker, chunk, num_chunks)
    f = pl.kernel(
        body,
        out_type=jax.ShapeDtypeStruct((n_meas,), jnp.float32),
        mesh=mesh,
        compiler_params=pltpu.CompilerParams(needs_layout_passes=False),
        scratch_types=[
            pltpu.VMEM((n_passes,), jnp.float32),
            pltpu.VMEM((chunk,), jnp.int32),
            pltpu.VMEM((chunk,), jnp.int32),
            pltpu.VMEM((chunk,), jnp.float32),
            pltpu.VMEM((chunk,), jnp.float32),
            pltpu.VMEM((chunk,), jnp.float32),
            pltpu.VMEM((chunk,), jnp.float32),
            pltpu.SemaphoreType.DMA,
            pltpu.SemaphoreType.DMA,
            pltpu.SemaphoreType.DMA,
            pltpu.SemaphoreType.DMA,
        ],
    )
    return f(range_km, sensor_params, contact_indices)


def kernel(range_km, sensor_params, contact_indices):
    idx = contact_indices.astype(jnp.int32)
    return _radar_bias_add(range_km, sensor_params, idx)


# speedup vs baseline: 641.4566x; 1.1029x over previous
import functools

import jax
import jax.numpy as jnp
from jax import lax
from jax.experimental import pallas as pl
from jax.experimental.pallas import tpu as pltpu
from jax.experimental.pallas import tpu_sc as plsc

N_LANES = 16
NUM_WORKERS = 32


def _gather_add_body(per_worker, chunk, num_chunks,
                     range_hbm, params_hbm, idx_hbm, out_hbm,
                     table_sh, table_v,
                     idx_v0, idx_v1, rng_v0, rng_v1, res_v0, res_v1,
                     isem0, isem1, osem0, osem1):
    s_id = lax.axis_index("s")
    wid = s_id * 2 + lax.axis_index("c")
    base = wid * per_worker
    idx_v = (idx_v0, idx_v1)
    rng_v = (rng_v0, rng_v1)
    res_v = (res_v0, res_v1)
    isems = (isem0, isem1)
    osems = (osem0, osem1)

    in_copies = {}
    out_copies = {}

    def issue_in(j):
        b = j & 1
        off = base + j * chunk
        in_copies[j] = (
            pltpu.async_copy(idx_hbm.at[pl.ds(off, chunk)],
                             idx_v[b], isems[b]),
            pltpu.async_copy(range_hbm.at[pl.ds(off, chunk)],
                             rng_v[b], isems[b]),
        )

    issue_in(0)

    @pl.when(s_id == 0)
    def _():
        pltpu.sync_copy(params_hbm, table_sh)

    plsc.subcore_barrier()
    pltpu.sync_copy(table_sh, table_v)
    for j in range(num_chunks):
        b = j & 1
        if j + 1 < num_chunks:
            issue_in(j + 1)
        ci, cr = in_copies.pop(j)
        ci.wait()
        cr.wait()
        if j >= 2:
            out_copies.pop(j - 2).wait()

        idx_b = idx_v[b]
        rng_b = rng_v[b]
        res_b = res_v[b]

        @plsc.parallel_loop(0, chunk, step=N_LANES, unroll=8)
        def _(i):
            s = pl.ds(i, N_LANES)
            vals = plsc.load_gather(table_v, [idx_b[s]])
            res_b[s] = rng_b[s] + vals

        out_copies[j] = pltpu.async_copy(
            res_v[b], out_hbm.at[pl.ds(base + j * chunk, chunk)], osems[b])

    for j in sorted(out_copies):
        out_copies[j].wait()


@jax.jit
def _radar_bias_add(range_km, sensor_params, contact_indices):
    n_meas = range_km.shape[0]
    n_passes = sensor_params.shape[0]
    assert n_meas % NUM_WORKERS == 0
    per_worker = n_meas // NUM_WORKERS
    chunk = 4096
    assert per_worker % chunk == 0
    num_chunks = per_worker // chunk

    mesh = plsc.VectorSubcoreMesh(core_axis_name="c", subcore_axis_name="s")
    body = functools.partial(_gather_add_body, per_worker, chunk, num_chunks)
    f = pl.kernel(
        body,
        out_type=jax.ShapeDtypeStruct((n_meas,), jnp.float32),
        mesh=mesh,
        compiler_params=pltpu.CompilerParams(needs_layout_passes=False),
        scratch_types=[
            pltpu.VMEM_SHARED((n_passes,), jnp.float32),
            pltpu.VMEM((n_passes,), jnp.float32),
            pltpu.VMEM((chunk,), jnp.int32),
            pltpu.VMEM((chunk,), jnp.int32),
            pltpu.VMEM((chunk,), jnp.float32),
            pltpu.VMEM((chunk,), jnp.float32),
            pltpu.VMEM((chunk,), jnp.float32),
            pltpu.VMEM((chunk,), jnp.float32),
            pltpu.SemaphoreType.DMA,
            pltpu.SemaphoreType.DMA,
            pltpu.SemaphoreType.DMA,
            pltpu.SemaphoreType.DMA,
        ],
    )
    return f(range_km, sensor_params, contact_indices)


def kernel(range_km, sensor_params, contact_indices):
    idx = contact_indices.astype(jnp.int32)
    return _radar_bias_add(range_km, sensor_params, idx)
